# Initial kernel scaffold; baseline (speedup 1.0000x reference)
import jax
import jax.numpy as jnp
from jax.experimental import pallas as pl


def kernel(x, Wg, W1, W2):
    raise NotImplementedError("write your pallas kernel here")


# speedup vs baseline: 85.1147x; 85.1147x over previous
import functools

import jax
import jax.numpy as jnp
from jax import lax
from jax.experimental import pallas as pl
from jax.experimental.pallas import tpu as pltpu

N_EMBD = 1024
N_INNER = 768
N_EXPERTS = 64
T = 2048
TILE_M = 128


def _routing_body(logits_ref, x_ref, xg_ref, pos_ref, w_ref, counts_ref):
    logits = logits_ref[...]
    m = jnp.max(logits, axis=1, keepdims=True)
    p = jnp.exp(logits - m)
    s = jnp.sum(p, axis=1, keepdims=True)
    probs = p / s
    w = jnp.max(probs, axis=1, keepdims=True)
    cols = lax.broadcasted_iota(jnp.int32, (T, N_EXPERTS), 1)
    e_sel = jnp.min(jnp.where(probs == w, cols, N_EXPERTS), axis=1, keepdims=True)
    onehot = (cols == e_sel).astype(jnp.float32)
    counts = jnp.sum(onehot, axis=0, keepdims=True)
    ri = lax.broadcasted_iota(jnp.int32, (T, T), 0)
    ci = lax.broadcasted_iota(jnp.int32, (T, T), 1)
    tril = (ci <= ri).astype(jnp.bfloat16)
    incl = jnp.dot(tril, onehot.astype(jnp.bfloat16),
                   preferred_element_type=jnp.float32)
    re = lax.broadcasted_iota(jnp.int32, (N_EXPERTS, N_EXPERTS), 0)
    ce = lax.broadcasted_iota(jnp.int32, (N_EXPERTS, N_EXPERTS), 1)
    upper = (re < ce).astype(jnp.float32)
    starts = jnp.dot(counts, upper, preferred_element_type=jnp.float32)
    posf = jnp.sum(onehot * (starts + incl - 1.0), axis=1, keepdims=True)
    posi = posf.astype(jnp.int32)
    ci_t = lax.broadcasted_iota(jnp.int32, (T, T), 1)
    perm = (ci_t == posi).astype(jnp.bfloat16)
    xg = lax.dot_general(perm, x_ref[...].astype(jnp.bfloat16),
                         (((0,), (0,)), ((), ())),
                         preferred_element_type=jnp.float32)
    xg_ref[...] = xg.astype(jnp.bfloat16)
    pos_ref[...] = posi
    w_ref[...] = w
    counts_ref[...] = counts.astype(jnp.int32)


def _gmm_body(starts_ref, xg_ref, w1_ref, w2_ref, out_ref):
    e = pl.program_id(0)
    s0 = starts_ref[e]
    s1 = starts_ref[e + 1]
    first = s0 - lax.rem(s0, 8)
    ntiles = lax.div(s1 - first + TILE_M - 1, TILE_M)
    w1 = w1_ref[0].astype(jnp.bfloat16)
    w2 = w2_ref[0].astype(jnp.bfloat16)

    def body(t, _):
        off = jnp.minimum(first + t * TILE_M, T - TILE_M)
        off = pl.multiple_of(off, 8)
        xt = xg_ref[pl.ds(off, TILE_M), :]
        h = lax.dot_general(xt, w1, (((1,), (1,)), ((), ())),
                            preferred_element_type=jnp.float32)
        h = 0.5 * h * (1.0 + lax.erf(h * (2.0 ** -0.5)))
        o = jnp.dot(h.astype(jnp.bfloat16), w2,
                    preferred_element_type=jnp.float32)
        rows = off + lax.broadcasted_iota(jnp.int32, (TILE_M, 1), 0)
        mask = (rows >= s0) & (rows < s1)
        cur = out_ref[pl.ds(off, TILE_M), :]
        out_ref[pl.ds(off, TILE_M), :] = jnp.where(mask, o, cur)
        return 0

    lax.fori_loop(0, ntiles, body, 0)


def _unperm_body(pos_ref, w_ref, outg_ref, out_ref):
    posi = pos_ref[...]
    ci = lax.broadcasted_iota(jnp.int32, (T, T), 1)
    gather = (ci == posi).astype(jnp.bfloat16)
    og = jnp.dot(gather, outg_ref[...].astype(jnp.bfloat16),
                 preferred_element_type=jnp.float32)
    out_ref[...] = og * w_ref[...]


def kernel(x, Wg, W1, W2):
    logits = x @ Wg.T

    xg, pos, w, counts = pl.pallas_call(
        _routing_body,
        out_shape=(
            jax.ShapeDtypeStruct((T, N_EMBD), jnp.bfloat16),
            jax.ShapeDtypeStruct((T, 1), jnp.int32),
            jax.ShapeDtypeStruct((T, 1), jnp.float32),
            jax.ShapeDtypeStruct((1, N_EXPERTS), jnp.int32),
        ),
    )(logits, x)

    starts = jnp.concatenate(
        [jnp.zeros((1,), jnp.int32), jnp.cumsum(counts[0]).astype(jnp.int32)])

    out_g = pl.pallas_call(
        _gmm_body,
        grid_spec=pltpu.PrefetchScalarGridSpec(
            num_scalar_prefetch=1,
            grid=(N_EXPERTS,),
            in_specs=[
                pl.BlockSpec((T, N_EMBD), lambda e, s: (0, 0)),
                pl.BlockSpec((1, N_INNER, N_EMBD), lambda e, s: (e, 0, 0)),
                pl.BlockSpec((1, N_INNER, N_EMBD), lambda e, s: (e, 0, 0)),
            ],
            out_specs=pl.BlockSpec((T, N_EMBD), lambda e, s: (0, 0)),
        ),
        out_shape=jax.ShapeDtypeStruct((T, N_EMBD), jnp.float32),
    )(starts, xg, W1, W2)

    out = pl.pallas_call(
        _unperm_body,
        out_shape=jax.ShapeDtypeStruct((T, N_EMBD), jnp.float32),
    )(pos, w, out_g)
    return out
